# Initial kernel scaffold; baseline (speedup 1.0000x reference)
import jax
import jax.numpy as jnp
from jax.experimental import pallas as pl


def kernel(x, edge_index, batch, W1, b1, W2, b2, Wg1, bg1, Wg2, bg2, Wo, bo):
    raise NotImplementedError("write your pallas kernel here")


# speedup vs baseline: 19.5042x; 19.5042x over previous
import functools

import jax
import jax.numpy as jnp
from jax import lax
from jax.experimental import pallas as pl
from jax.experimental.pallas import tpu as pltpu
from jax.experimental.pallas import tpu_sc as plsc

N = 10000
E = 320000
D = 128
NG = 64
NP = 10240
NC, NS = 2, 16
NW = NC * NS
K = 125
EPW = E // NW
NCH = EPW // K
ZC = 80
RPW = NP // NW
RPS = NP // NS
BR = 1024
GRID = NP // BR


def _sc_mesh():
    return plsc.VectorSubcoreMesh(core_axis_name="c", subcore_axis_name="s")


@functools.partial(
    pl.kernel,
    out_type=jax.ShapeDtypeStruct((NC, NP), jnp.float32),
    mesh=_sc_mesh(),
    scratch_types=[
        pltpu.VMEM((NCH, K), jnp.int32),
        pltpu.VMEM((RPS,), jnp.float32),
        pltpu.VMEM_SHARED((NP,), jnp.float32),
    ],
)
def _deg_kernel(dst2_hbm, out_hbm, dstv, const_v, deg_sh):
    cid = lax.axis_index("c")
    sid = lax.axis_index("s")
    wid = cid * NS + sid

    def fill(val):
        def body(i, _):
            const_v[pl.ds(i * 16, 16)] = jnp.full((16,), val, jnp.float32)
            return 0
        lax.fori_loop(0, RPS // 16, body, 0)

    fill(0.0)
    pltpu.sync_copy(const_v, deg_sh.at[pl.ds(sid * RPS, RPS)])
    pltpu.sync_copy(dst2_hbm.at[pl.ds(wid * NCH, NCH)], dstv)
    fill(1.0)
    plsc.subcore_barrier()

    def body(j, _):
        pltpu.sync_copy(const_v.at[pl.ds(0, K)], deg_sh.at[dstv.at[j]], add=True)
        return 0
    lax.fori_loop(0, NCH, body, 0)

    plsc.subcore_barrier()
    pltpu.sync_copy(deg_sh.at[pl.ds(sid * RPS, RPS)],
                    out_hbm.at[cid, pl.ds(sid * RPS, RPS)])


@functools.partial(
    pl.kernel,
    out_type=jax.ShapeDtypeStruct((NC, NP, D), jnp.float32),
    mesh=_sc_mesh(),
    scratch_types=[
        pltpu.VMEM((NCH, K), jnp.int32),
        pltpu.VMEM((NCH, K), jnp.int32),
        pltpu.VMEM((K, D), jnp.float32),
        pltpu.VMEM_SHARED((NP, D), jnp.float32),
        pltpu.SemaphoreType.DMA,
    ],
)
def _prop_kernel(m_hbm, src2_hbm, dst2_hbm, out_hbm, srcv, dstv, buf, p_sh, sem):
    cid = lax.axis_index("c")
    sid = lax.axis_index("s")
    wid = cid * NS + sid

    def zbody(i, _):
        r = i // (D // 16)
        c = (i % (D // 16)) * 16
        buf[r, pl.ds(c, 16)] = jnp.zeros((16,), jnp.float32)
        return 0
    lax.fori_loop(0, K * (D // 16), zbody, 0)
    for k in range(RPS // ZC):
        pltpu.sync_copy(buf.at[pl.ds(0, ZC)],
                        p_sh.at[pl.ds(sid * RPS + k * ZC, ZC)])

    pltpu.sync_copy(src2_hbm.at[pl.ds(wid * NCH, NCH)], srcv)
    pltpu.sync_copy(dst2_hbm.at[pl.ds(wid * NCH, NCH)], dstv)
    plsc.subcore_barrier()

    def body(j, _):
        pltpu.async_copy(m_hbm.at[srcv.at[j]], buf, sem).wait()
        pltpu.sync_copy(buf, p_sh.at[dstv.at[j]], add=True)
        return 0
    lax.fori_loop(0, NCH, body, 0)

    plsc.subcore_barrier()
    pltpu.sync_copy(p_sh.at[pl.ds(sid * RPS, RPS)],
                    out_hbm.at[cid, pl.ds(sid * RPS, RPS)])


def _dinv_block(degp):
    return lax.rsqrt(degp[0] + degp[1] + 1.0)


def _pre1_body(x_ref, w1_ref, degp_ref, m1_ref):
    dinv = _dinv_block(degp_ref[...])
    h = jnp.dot(x_ref[...], w1_ref[...], preferred_element_type=jnp.float32)
    m1_ref[...] = h * dinv[:, None]


def _mid_body(p1_ref, m1_ref, degp_ref, b1_ref, q_ref):
    dinv = _dinv_block(degp_ref[...])
    p = p1_ref[0] + p1_ref[1] + m1_ref[...]
    h1 = jax.nn.relu(p * dinv[:, None] + b1_ref[...][None, :])
    q_ref[...] = h1 * dinv[:, None]


def _post2_body(p2_ref, q_ref, degp_ref, w2_ref, b2_ref, batch_ref, gm_ref):
    i = pl.program_id(0)

    @pl.when(i == 0)
    def _():
        gm_ref[...] = jnp.zeros_like(gm_ref)

    dinv = _dinv_block(degp_ref[...])
    t = (p2_ref[0] + p2_ref[1] + q_ref[...]) * dinv[:, None]
    h2 = jax.nn.relu(
        jnp.dot(t, w2_ref[...], preferred_element_type=jnp.float32)
        + b2_ref[...][None, :])
    bb = batch_ref[0, 0]

    def gbody(g, _):
        row = jnp.max(jnp.where(bb[:, None] == g, h2, 0.0), axis=0,
                      keepdims=True)
        gm_ref[pl.ds(g, 1), :] = jnp.maximum(gm_ref[pl.ds(g, 1), :], row)
        return 0
    lax.fori_loop(0, NG, gbody, 0)


def _head_body(gm_ref, wg1_ref, bg1_ref, wg2_ref, bg2_ref, wo_ref, bo_ref,
               out_ref):
    z = jax.nn.relu(
        jnp.dot(gm_ref[...], wg1_ref[...], preferred_element_type=jnp.float32)
        + bg1_ref[...][None, :])
    z = jnp.dot(z, wg2_ref[...], preferred_element_type=jnp.float32)
    z = z + bg2_ref[...][None, :]
    out_ref[...] = (
        jnp.dot(z, wo_ref[...], preferred_element_type=jnp.float32)
        + bo_ref[...][None, :])


def _row_spec(shape):
    nd = len(shape)
    if nd == 2:
        return pl.BlockSpec((BR, shape[1]), lambda i: (i, 0))
    raise ValueError(shape)


def _full_spec(shape):
    nd = len(shape)
    return pl.BlockSpec(shape, (lambda *_: (0,) * nd))


def kernel(x, edge_index, batch, W1, b1, W2, b2, Wg1, bg1, Wg2, bg2, Wo, bo):
    src2 = edge_index[0].reshape(E // K, K)
    dst2 = edge_index[1].reshape(E // K, K)
    x_p = jnp.pad(x, ((0, NP - N), (0, 0)))
    batch3 = jnp.pad(batch, (0, NP - N), constant_values=NG).reshape(GRID, 1, BR)

    deg_parts = _deg_kernel(dst2)

    m1 = pl.pallas_call(
        _pre1_body,
        grid=(GRID,),
        in_specs=[
            _row_spec((NP, D)),
            _full_spec((D, D)),
            pl.BlockSpec((NC, BR), lambda i: (0, i)),
        ],
        out_specs=_row_spec((NP, D)),
        out_shape=jax.ShapeDtypeStruct((NP, D), jnp.float32),
    )(x_p, W1, deg_parts)

    p1 = _prop_kernel(m1, src2, dst2)

    q = pl.pallas_call(
        _mid_body,
        grid=(GRID,),
        in_specs=[
            pl.BlockSpec((NC, BR, D), lambda i: (0, i, 0)),
            _row_spec((NP, D)),
            pl.BlockSpec((NC, BR), lambda i: (0, i)),
            _full_spec((D,)),
        ],
        out_specs=_row_spec((NP, D)),
        out_shape=jax.ShapeDtypeStruct((NP, D), jnp.float32),
    )(p1, m1, deg_parts, b1)

    p2 = _prop_kernel(q, src2, dst2)

    gm = pl.pallas_call(
        _post2_body,
        grid=(GRID,),
        in_specs=[
            pl.BlockSpec((NC, BR, D), lambda i: (0, i, 0)),
            _row_spec((NP, D)),
            pl.BlockSpec((NC, BR), lambda i: (0, i)),
            _full_spec((D, 2 * D)),
            _full_spec((2 * D,)),
            pl.BlockSpec((1, 1, BR), lambda i: (i, 0, 0)),
        ],
        out_specs=_full_spec((NG, 2 * D)),
        out_shape=jax.ShapeDtypeStruct((NG, 2 * D), jnp.float32),
    )(p2, q, deg_parts, W2, b2, batch3)

    out = pl.pallas_call(
        _head_body,
        in_specs=[
            _full_spec((NG, 2 * D)),
            _full_spec((2 * D, 1024)),
            _full_spec((1024,)),
            _full_spec((1024, D)),
            _full_spec((D,)),
            _full_spec((D, 1)),
            _full_spec((1,)),
        ],
        out_specs=_full_spec((NG, 1)),
        out_shape=jax.ShapeDtypeStruct((NG, 1), jnp.float32),
    )(gm, Wg1, bg1, Wg2, bg2, Wo, bo)

    return out
